# baseline (speedup 1.0000x reference)
import functools

import jax
import jax.numpy as jnp
from jax import lax
from jax.experimental import pallas as pl
from jax.experimental.pallas import tpu as pltpu
from jax.experimental.pallas import tpu_sc as plsc

N_CLASSES = 1000
BATCH = 1024
SEQ = 20
D = N_CLASSES
DT = D // 8

NUM_CORES = 2
NUM_SUBCORES = 16
NW = NUM_CORES * NUM_SUBCORES
B_PER_W = BATCH // NW
BH = B_PER_W // 2
TB_MINOR = BH + 1
NSTEP = 63

_mesh = plsc.VectorSubcoreMesh(core_axis_name="c", subcore_axis_name="s")


@functools.partial(
    pl.kernel,
    out_type=jax.ShapeDtypeStruct((SEQ, DT, 8, 8, 128), jnp.float32),
    mesh=_mesh,
    compiler_params=pltpu.CompilerParams(
        use_tc_tiling_on_sc=False, needs_layout_passes=False),
    scratch_types=[
        pltpu.VMEM((SEQ, B_PER_W), jnp.int32),
        pltpu.VMEM((BH, D), jnp.float32),
        pltpu.VMEM((BH, D), jnp.float32),
        pltpu.VMEM((BH, D), jnp.float32),
        pltpu.VMEM((BH, D), jnp.float32),
        pltpu.VMEM((DT, 8, TB_MINOR), jnp.float32),
        pltpu.VMEM((DT, 8, TB_MINOR), jnp.float32),
        pltpu.SemaphoreType.DMA,
        pltpu.SemaphoreType.DMA,
        pltpu.SemaphoreType.DMA,
        pltpu.SemaphoreType.DMA,
        pltpu.SemaphoreType.DMA,
        pltpu.SemaphoreType.DMA,
    ],
)
def _gather_t(idx_hbm, w_hbm, out_hbm, idx_t, ga0, gb0, ga1, gb1, tb0, tb1,
              sa0, sb0, sa1, sb1, osem0, osem1):
    wid = lax.axis_index("s") * NUM_CORES + lax.axis_index("c")
    pltpu.sync_copy(idx_hbm.at[:, pl.ds(wid * B_PER_W, B_PER_W)], idx_t)
    bt = wid // 4
    off = (wid % 4) * B_PER_W

    iota16 = lax.iota(jnp.int32, 16)
    bconsts = [jnp.full((16,), b, jnp.int32) for b in range(BH)]

    def transpose_b(gb, tb):
        @plsc.parallel_loop(0, NSTEP, unroll=2)
        def _step(g):
            d0 = lax.min(g * 16, D - 16)
            dvec = iota16 + d0
            dt_v = dvec // 8
            dr_v = dvec % 8
            for b0 in range(0, BH, 8):
                xs = [gb[b0 + i, pl.ds(d0, 16)] for i in range(8)]
                for i in range(8):
                    plsc.store_scatter(
                        tb, [dt_v, dr_v, bconsts[b0 + i]], xs[i])

    def g_start(s, h, gb, gsem):
        pltpu.async_copy(
            w_hbm.at[idx_t.at[s, pl.ds(h * BH, BH)]], gb, gsem)

    def g_wait(gb, gsem):
        pltpu.make_async_copy(
            w_hbm.at[idx_t.at[0, pl.ds(0, BH)]], gb, gsem).wait()

    def out_slice(s, h):
        return out_hbm.at[s, :, bt, :, pl.ds(off + h * BH, BH)]

    def w_wait(tb, h, osem):
        pltpu.make_async_copy(
            tb.at[:, :, pl.ds(0, BH)], out_slice(0, h), osem).wait()

    def w_start(tb, s, h, osem):
        pltpu.async_copy(tb.at[:, :, pl.ds(0, BH)], out_slice(s, h), osem)

    def half(s, h, gb, gsem, tb, osem, first=False, refill=True):
        g_wait(gb, gsem)
        if not first:
            w_wait(tb, h, osem)
        transpose_b(gb, tb)
        w_start(tb, s, h, osem)
        if refill:
            g_start(s + 2, h, gb, gsem)

    def unit(s, ga, sa, gbuf, sb, first=False, refill=True):
        half(s, 0, ga, sa, tb0, osem0, first=first, refill=refill)
        half(s, 1, gbuf, sb, tb1, osem1, first=first, refill=refill)

    g_start(0, 0, ga0, sa0)
    g_start(0, 1, gb0, sb0)
    g_start(1, 0, ga1, sa1)
    g_start(1, 1, gb1, sb1)

    unit(0, ga0, sa0, gb0, sb0, first=True)
    unit(1, ga1, sa1, gb1, sb1)

    def body(k, carry):
        s0 = 2 * k
        unit(s0, ga0, sa0, gb0, sb0)
        unit(s0 + 1, ga1, sa1, gb1, sb1)
        return carry
    lax.fori_loop(1, SEQ // 2 - 1, body, 0)

    unit(SEQ - 2, ga0, sa0, gb0, sb0, refill=False)
    unit(SEQ - 1, ga1, sa1, gb1, sb1, refill=False)
    w_wait(tb0, 0, osem0)
    w_wait(tb1, 1, osem1)


def kernel(inp, hidden, W):
    out5 = _gather_t(inp.T.astype(jnp.int32), W)
    out = out5.transpose(2, 4, 0, 1, 3).reshape(BATCH, SEQ, D)
    return (out, hidden)
